# baseline (speedup 1.0000x reference)
import functools
import math

import jax
import jax.numpy as jnp
from jax import lax
from jax.experimental import pallas as pl
from jax.experimental.pallas import tpu as pltpu
from jax.experimental.pallas import tpu_sc as plsc

D_MODEL = 64
SCALE = math.sqrt(D_MODEL)
LANES = 16

NUM_CORES = 2
NUM_SUBCORES = 16
NW = NUM_CORES * NUM_SUBCORES

VOCAB = 1000000
SEQ = 200
BATCH = 4096
BW = BATCH // NW
BW_PAD = BW + 1
PAIR = 2 * D_MODEL

CB = 256
CBH = CB // 2
CBH_PAD = PAIR + 1
NBLK = VOCAB // CB
NB_W = NBLK // NW
TAIL = VOCAB - NBLK * CB
TAIL0 = NBLK * CB

_mesh = plsc.VectorSubcoreMesh(core_axis_name="c", subcore_axis_name="s")


@functools.partial(
    pl.kernel,
    out_type=jax.ShapeDtypeStruct((VOCAB // 2, PAIR), jnp.float32),
    mesh=_mesh,
    scratch_types=[
        [pltpu.VMEM((D_MODEL, CB), jnp.float32) for _ in range(2)],
        [pltpu.VMEM((CBH, CBH_PAD), jnp.float32) for _ in range(2)],
        [pltpu.SemaphoreType.DMA for _ in range(2)],
        [pltpu.SemaphoreType.DMA for _ in range(2)],
    ],
    compiler_params=pltpu.CompilerParams(
        use_tc_tiling_on_sc=True, needs_layout_passes=False
    ),
)
def _compact_pairs(tt_hbm, tailp_hbm, pairs_hbm, inb, outb, sem_i, sem_o):
    wid = lax.axis_index("s") * NUM_CORES + lax.axis_index("c")
    lane = lax.iota(jnp.int32, LANES)

    def fire_in(g, b):
        pltpu.async_copy(tt_hbm.at[:, pl.ds(g * CB, CB)], inb[b], sem_i[b])

    def wait_in(b):
        pltpu.make_async_copy(
            tt_hbm.at[:, pl.ds(0, CB)], inb[b], sem_i[b]
        ).wait()

    def store_block(b, g):
        pltpu.async_copy(
            outb[b].at[:, pl.ds(0, PAIR)],
            pairs_hbm.at[pl.ds(g * CBH, CBH)],
            sem_o[b],
        )

    def wait_out(b):
        pltpu.make_async_copy(
            outb[b].at[:, pl.ds(0, PAIR)],
            pairs_hbm.at[pl.ds(0, CBH)],
            sem_o[b],
        ).wait()

    def transpose_block(b):
        for j in range(CB // LANES):
            vcols = lane + (j * LANES)
            rowv = lax.shift_right_logical(vcols, 1)
            colb = (vcols & 1) * D_MODEL

            @plsc.parallel_loop(0, D_MODEL, unroll=8)
            def _(c):
                v = inb[b][c, pl.ds(j * LANES, LANES)]
                plsc.store_scatter(outb[b], [rowv, colb + c], v)

    fire_in(wid, 0)

    def body(k, carry):
        for b in range(2):
            kk = k * 2 + b
            g = wid + NW * kk

            @pl.when(kk + 1 < NB_W)
            def _():
                fire_in(wid + NW * (kk + 1), 1 - b)

            wait_in(b)

            @pl.when(kk >= 2)
            def _():
                wait_out(b)

            transpose_block(b)
            store_block(b, g)
        return carry

    lax.fori_loop(0, NB_W // 2, body, 0)

    for b in range(2):
        wait_out(b)

    @pl.when(wid < 2)
    def _():
        g = NB_W * NW + wid
        fire_in(g, 0)
        wait_in(0)
        transpose_block(0)
        pltpu.sync_copy(
            outb[0].at[:, pl.ds(0, PAIR)], pairs_hbm.at[pl.ds(g * CBH, CBH)]
        )

    @pl.when(wid == 2)
    def _():
        pltpu.sync_copy(tailp_hbm, outb[0].at[pl.ds(0, TAIL // 2), pl.ds(0, PAIR)])
        pltpu.sync_copy(
            outb[0].at[pl.ds(0, TAIL // 2), pl.ds(0, PAIR)],
            pairs_hbm.at[pl.ds(TAIL0 // 2, TAIL // 2)],
        )


NBUF = 4
FIRE_AHEAD = 2


@functools.partial(
    pl.kernel,
    out_type=jax.ShapeDtypeStruct((SEQ, D_MODEL, BATCH), jnp.float32),
    mesh=_mesh,
    scratch_types=[
        pltpu.VMEM((SEQ, BW), jnp.int32),
        [pltpu.VMEM((BW, D_MODEL), jnp.float32) for _ in range(NBUF)],
        [pltpu.VMEM((D_MODEL, BW_PAD), jnp.float32) for _ in range(NBUF)],
        [pltpu.SemaphoreType.DMA for _ in range(NBUF)],
        [pltpu.SemaphoreType.DMA for _ in range(NBUF)],
    ],
    compiler_params=pltpu.CompilerParams(
        use_tc_tiling_on_sc=False, needs_layout_passes=False
    ),
)
def _emb_lookup(xt_hbm, table_hbm, out_hbm, idx_v, rows, trans, sem_g, sem_s):
    wid = lax.axis_index("s") * NUM_CORES + lax.axis_index("c")
    bbase = wid * BW

    pltpu.sync_copy(xt_hbm.at[:, pl.ds(bbase, BW)], idx_v)

    lane = lax.iota(jnp.int32, LANES)

    def fire_gather(t, b):
        pltpu.async_copy(table_hbm.at[idx_v.at[t]], rows[b], sem_g[b])

    def wait_gather(b):
        pltpu.make_async_copy(table_hbm.at[idx_v.at[0]], rows[b], sem_g[b]).wait()

    def wait_store(b):
        pltpu.make_async_copy(
            trans[b].at[:, pl.ds(0, BW)], out_hbm.at[0, :, pl.ds(0, BW)], sem_s[b]
        ).wait()

    for t in range(FIRE_AHEAD):
        fire_gather(t, t)

    def outer(t0, carry):
        for b in range(NBUF):
            t = t0 * NBUF + b
            fb = (b + FIRE_AHEAD) % NBUF

            @pl.when(t + FIRE_AHEAD < SEQ)
            def _():
                fire_gather(t + FIRE_AHEAD, fb)

            wait_gather(b)

            @pl.when(t >= NBUF)
            def _():
                wait_store(b)

            @plsc.parallel_loop(0, BW, unroll=4)
            def _(i):
                coli = jnp.full((LANES,), i, dtype=jnp.int32)
                for q in range(D_MODEL // LANES):
                    v = rows[b][i, pl.ds(q * LANES, LANES)] * SCALE
                    plsc.store_scatter(trans[b], [lane + (q * LANES), coli], v)

            pltpu.async_copy(
                trans[b].at[:, pl.ds(0, BW)],
                out_hbm.at[t, :, pl.ds(bbase, BW)],
                sem_s[b],
            )
        return carry

    lax.fori_loop(0, SEQ // NBUF, outer, 0)

    for b in range(NBUF):
        wait_store(b)


def kernel(x, table):
    xt = jnp.transpose(x.astype(jnp.int32))
    tt = jnp.transpose(table)
    tailp = jnp.reshape(
        lax.slice(table, (TAIL0, 0), (VOCAB, D_MODEL)), (TAIL // 2, PAIR)
    )
    pairs = _compact_pairs(tt, tailp)
    out = _emb_lookup(xt, jnp.reshape(pairs, (VOCAB, D_MODEL)))
    return jnp.transpose(out, (2, 0, 1))


# speedup vs baseline: 1.2925x; 1.2925x over previous
import functools
import math

import jax
import jax.numpy as jnp
from jax import lax
from jax.experimental import pallas as pl
from jax.experimental.pallas import tpu as pltpu
from jax.experimental.pallas import tpu_sc as plsc

D_MODEL = 64
SCALE = math.sqrt(D_MODEL)
LANES = 16

NUM_CORES = 2
NUM_SUBCORES = 16
NW = NUM_CORES * NUM_SUBCORES

SEQ = 200
BATCH = 4096
BW = BATCH // NW
TR = SEQ // 8
BW_PAD = BW + 1
NBUF = 4
FIRE_AHEAD = 2

_mesh = plsc.VectorSubcoreMesh(core_axis_name="c", subcore_axis_name="s")


@functools.partial(
    pl.kernel,
    out_type=jax.ShapeDtypeStruct((SEQ, D_MODEL, BATCH), jnp.float32),
    mesh=_mesh,
    scratch_types=[
        pltpu.VMEM((TR, 8, BW), jnp.int32),
        [pltpu.VMEM((BW, D_MODEL), jnp.float32) for _ in range(NBUF)],
        [pltpu.VMEM((D_MODEL, BW_PAD), jnp.float32) for _ in range(NBUF)],
        [pltpu.SemaphoreType.DMA for _ in range(NBUF)],
        [pltpu.SemaphoreType.DMA for _ in range(NBUF)],
    ],
    compiler_params=pltpu.CompilerParams(
        use_tc_tiling_on_sc=False,
        needs_layout_passes=False,
        skip_device_barrier=True,
        disable_bounds_checks=True
    ),
)
def _emb_lookup(xt_hbm, table_hbm, out_hbm, idx_v, rows, trans, sem_g, sem_s):
    wid = lax.axis_index("s") * NUM_CORES + lax.axis_index("c")
    bbase = wid * BW

    pltpu.sync_copy(xt_hbm.at[:, wid], idx_v)

    lane = lax.iota(jnp.int32, LANES)

    def idx_slice(t):
        return idx_v.at[lax.shift_right_logical(t, 3), t & 7]

    def fire_gather(t, b):
        pltpu.async_copy(table_hbm.at[idx_slice(t)], rows[b], sem_g[b])

    def wait_gather(b):
        pltpu.make_async_copy(
            table_hbm.at[idx_v.at[0, 0]], rows[b], sem_g[b]
        ).wait()

    def wait_store(b):
        pltpu.make_async_copy(
            trans[b].at[:, pl.ds(0, BW)], out_hbm.at[0, :, pl.ds(0, BW)], sem_s[b]
        ).wait()

    for t in range(FIRE_AHEAD):
        pltpu.async_copy(table_hbm.at[idx_v.at[0, t]], rows[t], sem_g[t])

    def outer(t0, carry):
        for b in range(NBUF):
            t = t0 * NBUF + b
            fb = (b + FIRE_AHEAD) % NBUF

            @pl.when(t + FIRE_AHEAD < SEQ)
            def _():
                fire_gather(t + FIRE_AHEAD, fb)

            wait_gather(b)

            @pl.when(t >= NBUF)
            def _():
                wait_store(b)

            @plsc.parallel_loop(0, BW, unroll=4)
            def _(i):
                coli = jnp.full((LANES,), i, dtype=jnp.int32)
                for q in range(D_MODEL // LANES):
                    v = rows[b][i, pl.ds(q * LANES, LANES)] * SCALE
                    plsc.store_scatter(trans[b], [lane + (q * LANES), coli], v)

            pltpu.async_copy(
                trans[b].at[:, pl.ds(0, BW)],
                out_hbm.at[t, :, pl.ds(bbase, BW)],
                sem_s[b],
            )
        return carry

    lax.fori_loop(0, SEQ // NBUF, outer, 0)

    for b in range(NBUF):
        wait_store(b)


def kernel(x, table):
    xt4 = (
        jnp.transpose(x.astype(jnp.int32))
        .reshape(TR, 8, NW, BW)
        .transpose(0, 2, 1, 3)
    )
    out = _emb_lookup(xt4, table)
    return jnp.transpose(out, (2, 0, 1))
